# Initial kernel scaffold; baseline (speedup 1.0000x reference)
import functools

import jax
import jax.numpy as jnp
from jax import lax
from jax.experimental import pallas as pl
from jax.experimental.pallas import tpu as pltpu
from jax.experimental.pallas import tpu_sc as plsc

NUSR = 10000
NTOT = 50000
NEDGE = 800000
D = 64
NPAD = 51200
R_SC = NPAD // 2
NS = 16
NC = 2
R_TILE = R_SC // NS
CH = 128
ZROWS = 200
BGATH = 3072
GPW = BGATH // (NC * NS)

_mesh = plsc.VectorSubcoreMesh(
    core_axis_name="c", subcore_axis_name="s", num_cores=NC, num_subcores=NS)


@functools.partial(
    pl.kernel,
    out_type=jax.ShapeDtypeStruct((NPAD, D), jnp.float32),
    mesh=_mesh,
    scratch_types=[
        pltpu.VMEM_SHARED((R_SC, D), jnp.float32),
        pltpu.VMEM((16,), jnp.int32),
        pltpu.VMEM((CH,), jnp.int32),
        pltpu.VMEM((CH,), jnp.float32),
        pltpu.VMEM((CH,), jnp.int32),
        pltpu.VMEM((CH,), jnp.int32),
        pltpu.VMEM((CH, D), jnp.float32),
        pltpu.VMEM((ZROWS, D), jnp.float32),
        pltpu.SemaphoreType.DMA,
    ],
)
def _spmm(ego_hbm, col_hbm, val_hbm, row_hbm, bounds_hbm, out_hbm,
          acc, bbuf, colb, valb, rowb, rloc, gbuf, zbuf, sem):
    c = lax.axis_index("c")
    s = lax.axis_index("s")
    pltpu.sync_copy(bounds_hbm, bbuf)
    e_lo = bbuf[c]
    e_hi = bbuf[c + 1]
    base_row = c * R_SC

    zero16 = jnp.zeros((16,), jnp.float32)

    def zrow(i, carry):
        for k in range(D // 16):
            zbuf[i, pl.ds(k * 16, 16)] = zero16
        return carry

    lax.fori_loop(0, ZROWS, zrow, 0)
    for t in range(R_TILE // ZROWS):
        pltpu.sync_copy(zbuf, acc.at[pl.ds(s * R_TILE + t * ZROWS, ZROWS)])
    plsc.subcore_barrier()

    c0 = e_lo // CH
    c1 = (e_hi + CH - 1) // CH
    nchunks = c1 - c0
    q = nchunks // NS
    rem = nchunks % NS
    start = c0 + s * q + jnp.minimum(s, rem)
    cnt = q + jnp.where(s < rem, 1, 0)

    def chunk(i, carry):
        off = pl.multiple_of((start + i) * CH, CH)
        pltpu.sync_copy(col_hbm.at[pl.ds(off, CH)], colb)
        pltpu.sync_copy(val_hbm.at[pl.ds(off, CH)], valb)
        pltpu.sync_copy(row_hbm.at[pl.ds(off, CH)], rowb)
        pltpu.async_copy(ego_hbm.at[colb], gbuf, sem).wait()
        for j in range(CH // 16):
            ids = off + j * 16 + lax.iota(jnp.int32, 16)
            keep = (ids >= e_lo) & (ids < e_hi)
            vv = jnp.where(keep, valb[pl.ds(j * 16, 16)], 0.0)
            rl = rowb[pl.ds(j * 16, 16)] - base_row
            rl = jnp.minimum(jnp.maximum(rl, 0), R_SC - 1)
            valb[pl.ds(j * 16, 16)] = vv
            rloc[pl.ds(j * 16, 16)] = rl

        def scale(e, carry2):
            v = valb[e]
            for k in range(D // 16):
                gbuf[e, pl.ds(k * 16, 16)] = gbuf[e, pl.ds(k * 16, 16)] * v
            return carry2

        lax.fori_loop(0, CH, scale, 0)
        pltpu.sync_copy(gbuf, acc.at[rloc], add=True)
        return carry

    lax.fori_loop(0, cnt, chunk, 0)
    plsc.subcore_barrier()
    pltpu.sync_copy(acc.at[pl.ds(s * R_TILE, R_TILE)],
                    out_hbm.at[pl.ds(base_row + s * R_TILE, R_TILE)])


def _dense_body(side_ref, ego_ref, wgc_ref, bgc_ref, wbi_ref, bbi_ref,
                next_ref, norm_ref):
    sd = side_ref[...]
    eg = ego_ref[...]
    t = jnp.dot(sd, wgc_ref[...], preferred_element_type=jnp.float32)
    t = t + jnp.dot(eg * sd, wbi_ref[...], preferred_element_type=jnp.float32)
    t = t + bgc_ref[...] + bbi_ref[...]
    a = jnp.where(t >= 0, t, 0.2 * t)
    next_ref[...] = a
    nrm = jnp.sqrt(jnp.sum(a * a, axis=1, keepdims=True))
    norm_ref[...] = a / jnp.maximum(nrm, 1e-12)


_DBLK = 1024


def _dense(side, ego, wgc, bgc, wbi, bbi):
    grid = (NPAD // _DBLK,)
    row_spec = pl.BlockSpec((_DBLK, D), lambda i: (i, 0))
    full = pl.BlockSpec((D, D), lambda i: (0, 0))
    bias = pl.BlockSpec((1, D), lambda i: (0, 0))
    return pl.pallas_call(
        _dense_body,
        grid=grid,
        in_specs=[row_spec, row_spec, full, bias, full, bias],
        out_specs=[row_spec, row_spec],
        out_shape=[jax.ShapeDtypeStruct((NPAD, D), jnp.float32),
                   jax.ShapeDtypeStruct((NPAD, D), jnp.float32)],
    )(side, ego, wgc, bgc, wbi, bbi)


@functools.partial(
    pl.kernel,
    out_type=[jax.ShapeDtypeStruct((BGATH, D), jnp.float32)] * 4,
    mesh=_mesh,
    scratch_types=[
        pltpu.VMEM((GPW,), jnp.int32),
        pltpu.VMEM((GPW, D), jnp.float32),
        pltpu.SemaphoreType.DMA,
    ],
)
def _gather4(t0, t1, t2, t3, idx_hbm, o0, o1, o2, o3, idxb, rows, sem):
    c = lax.axis_index("c")
    s = lax.axis_index("s")
    base = (s * NC + c) * GPW
    pltpu.sync_copy(idx_hbm.at[pl.ds(base, GPW)], idxb)
    for tbl, out in ((t0, o0), (t1, o1), (t2, o2), (t3, o3)):
        pltpu.async_copy(tbl.at[idxb], rows, sem).wait()
        pltpu.sync_copy(rows, out.at[pl.ds(base, GPW)])


def kernel(user_emb, item_emb,
           W_gc_0, b_gc_0, W_bi_0, b_bi_0,
           W_gc_1, b_gc_1, W_bi_1, b_bi_1,
           W_gc_2, b_gc_2, W_bi_2, b_bi_2,
           adj_row, adj_col, adj_val,
           users, pos_items, neg_items):
    ego0 = jnp.concatenate(
        [user_emb, item_emb,
         jnp.zeros((NPAD - NTOT, D), jnp.float32)], axis=0)
    row = adj_row.astype(jnp.int32)
    col = adj_col.astype(jnp.int32)
    e_mid = jnp.searchsorted(row, R_SC, side="left").astype(jnp.int32)
    bounds = jnp.zeros((16,), jnp.int32)
    bounds = bounds.at[1].set(e_mid).at[2].set(NEDGE)

    params = [(W_gc_0, b_gc_0, W_bi_0, b_bi_0),
              (W_gc_1, b_gc_1, W_bi_1, b_bi_1),
              (W_gc_2, b_gc_2, W_bi_2, b_bi_2)]
    ego = ego0
    norms = []
    for (wgc, bgc, wbi, bbi) in params:
        side = _spmm(ego, col, adj_val, row, bounds)
        ego, nrm = _dense(side, ego, wgc, bgc, wbi, bbi)
        norms.append(nrm)

    idx = jnp.concatenate([users.astype(jnp.int32),
                           pos_items.astype(jnp.int32) + NUSR,
                           neg_items.astype(jnp.int32) + NUSR])
    o0, o1, o2, o3 = _gather4(ego0, norms[0], norms[1], norms[2], idx)
    allg = jnp.concatenate([o0, o1, o2, o3], axis=1)
    return (allg[:1024], allg[1024:2048], allg[2048:])


# speedup vs baseline: 3.5402x; 3.5402x over previous
import functools

import jax
import jax.numpy as jnp
from jax import lax
from jax.experimental import pallas as pl
from jax.experimental.pallas import tpu as pltpu
from jax.experimental.pallas import tpu_sc as plsc

NUSR = 10000
NTOT = 50000
NEDGE = 800000
D = 64
NPAD = 51200
R_SC = NPAD // 2
NS = 16
NC = 2
R_TILE = R_SC // NS
CH = 128
ZROWS = 200
BGATH = 3072
GPW = BGATH // (NC * NS)

_mesh = plsc.VectorSubcoreMesh(
    core_axis_name="c", subcore_axis_name="s", num_cores=NC, num_subcores=NS)


@functools.partial(
    pl.kernel,
    out_type=jax.ShapeDtypeStruct((NPAD, D), jnp.float32),
    mesh=_mesh,
    scratch_types=[
        pltpu.VMEM_SHARED((R_SC, D), jnp.float32),
        pltpu.VMEM((16,), jnp.int32),
        pltpu.VMEM((CH,), jnp.int32),
        pltpu.VMEM((CH,), jnp.float32),
        pltpu.VMEM((CH,), jnp.int32),
        pltpu.VMEM((CH,), jnp.int32),
        pltpu.VMEM((CH, D), jnp.float32),
        pltpu.VMEM((ZROWS, D), jnp.float32),
        pltpu.SemaphoreType.DMA,
    ],
    compiler_params=pltpu.CompilerParams(use_tc_tiling_on_sc=False),
)
def _spmm(ego_hbm, col_hbm, val_hbm, row_hbm, bounds_hbm, out_hbm,
          acc, bbuf, colb, valb, rowb, rloc, gbuf, zbuf, sem):
    c = lax.axis_index("c")
    s = lax.axis_index("s")
    pltpu.sync_copy(bounds_hbm, bbuf)
    bv = bbuf[pl.ds(0, 16)]
    e_lo = jnp.where(c == 0, bv[0], bv[1])
    e_hi = jnp.where(c == 0, bv[1], bv[2])
    base_row = c * R_SC

    zero16 = jnp.zeros((16,), jnp.float32)

    def zrow(i, carry):
        for k in range(D // 16):
            zbuf[i, pl.ds(k * 16, 16)] = zero16
        return carry

    lax.fori_loop(0, ZROWS, zrow, 0)
    for t in range(R_TILE // ZROWS):
        pltpu.sync_copy(zbuf, acc.at[pl.ds(s * R_TILE + t * ZROWS, ZROWS)])
    plsc.subcore_barrier()

    c0 = e_lo // CH
    c1 = (e_hi + CH - 1) // CH
    nchunks = c1 - c0
    q = nchunks // NS
    rem = nchunks % NS
    start = c0 + s * q + jnp.minimum(s, rem)
    cnt = q + jnp.where(s < rem, 1, 0)

    def chunk(i, carry):
        off = pl.multiple_of((start + i) * CH, CH)
        pltpu.sync_copy(col_hbm.at[pl.ds(off, CH)], colb)
        pltpu.sync_copy(val_hbm.at[pl.ds(off, CH)], valb)
        pltpu.sync_copy(row_hbm.at[pl.ds(off, CH)], rowb)
        pltpu.async_copy(ego_hbm.at[colb], gbuf, sem).wait()
        for j in range(CH // 16):
            ids = off + j * 16 + lax.iota(jnp.int32, 16)
            keep = (ids >= e_lo) & (ids < e_hi)
            vv = jnp.where(keep, valb[pl.ds(j * 16, 16)], 0.0)
            rl = rowb[pl.ds(j * 16, 16)] - base_row
            rl = jnp.minimum(jnp.maximum(rl, 0), R_SC - 1)
            valb[pl.ds(j * 16, 16)] = vv
            rloc[pl.ds(j * 16, 16)] = rl

        def scale(j, carry2):
            vv = valb[pl.ds(j * 16, 16)]
            for l in range(16):
                v = vv[l]
                e = j * 16 + l
                for k in range(D // 16):
                    gbuf[e, pl.ds(k * 16, 16)] = gbuf[e, pl.ds(k * 16, 16)] * v
            return carry2

        lax.fori_loop(0, CH // 16, scale, 0)
        pltpu.sync_copy(gbuf, acc.at[rloc], add=True)
        return carry

    lax.fori_loop(0, cnt, chunk, 0)
    plsc.subcore_barrier()
    pltpu.sync_copy(acc.at[pl.ds(s * R_TILE, R_TILE)],
                    out_hbm.at[pl.ds(base_row + s * R_TILE, R_TILE)])


def _dense_body(side_ref, ego_ref, wgc_ref, bgc_ref, wbi_ref, bbi_ref,
                next_ref, norm_ref):
    sd = side_ref[...]
    eg = ego_ref[...]
    t = jnp.dot(sd, wgc_ref[...], preferred_element_type=jnp.float32)
    t = t + jnp.dot(eg * sd, wbi_ref[...], preferred_element_type=jnp.float32)
    t = t + bgc_ref[...] + bbi_ref[...]
    a = jnp.where(t >= 0, t, 0.2 * t)
    next_ref[...] = a
    nrm = jnp.sqrt(jnp.sum(a * a, axis=1, keepdims=True))
    norm_ref[...] = a / jnp.maximum(nrm, 1e-12)


_DBLK = 1024


def _dense(side, ego, wgc, bgc, wbi, bbi):
    grid = (NPAD // _DBLK,)
    row_spec = pl.BlockSpec((_DBLK, D), lambda i: (i, 0))
    full = pl.BlockSpec((D, D), lambda i: (0, 0))
    bias = pl.BlockSpec((1, D), lambda i: (0, 0))
    return pl.pallas_call(
        _dense_body,
        grid=grid,
        in_specs=[row_spec, row_spec, full, bias, full, bias],
        out_specs=[row_spec, row_spec],
        out_shape=[jax.ShapeDtypeStruct((NPAD, D), jnp.float32),
                   jax.ShapeDtypeStruct((NPAD, D), jnp.float32)],
    )(side, ego, wgc, bgc, wbi, bbi)


@functools.partial(
    pl.kernel,
    out_type=[jax.ShapeDtypeStruct((BGATH, D), jnp.float32)] * 4,
    mesh=_mesh,
    scratch_types=[
        pltpu.VMEM((GPW,), jnp.int32),
        pltpu.VMEM((GPW, D), jnp.float32),
        pltpu.SemaphoreType.DMA,
    ],
    compiler_params=pltpu.CompilerParams(use_tc_tiling_on_sc=False),
)
def _gather4(t0, t1, t2, t3, idx_hbm, o0, o1, o2, o3, idxb, rows, sem):
    c = lax.axis_index("c")
    s = lax.axis_index("s")
    base = (s * NC + c) * GPW
    pltpu.sync_copy(idx_hbm.at[pl.ds(base, GPW)], idxb)
    for tbl, out in ((t0, o0), (t1, o1), (t2, o2), (t3, o3)):
        pltpu.async_copy(tbl.at[idxb], rows, sem).wait()
        pltpu.sync_copy(rows, out.at[pl.ds(base, GPW)])


def kernel(user_emb, item_emb,
           W_gc_0, b_gc_0, W_bi_0, b_bi_0,
           W_gc_1, b_gc_1, W_bi_1, b_bi_1,
           W_gc_2, b_gc_2, W_bi_2, b_bi_2,
           adj_row, adj_col, adj_val,
           users, pos_items, neg_items):
    ego0 = jnp.concatenate(
        [user_emb, item_emb,
         jnp.zeros((NPAD - NTOT, D), jnp.float32)], axis=0)
    row = adj_row.astype(jnp.int32)
    col = adj_col.astype(jnp.int32)
    e_mid = jnp.searchsorted(row, R_SC, side="left").astype(jnp.int32)
    bounds = jnp.zeros((16,), jnp.int32)
    bounds = bounds.at[1].set(e_mid).at[2].set(NEDGE)

    params = [(W_gc_0, b_gc_0, W_bi_0, b_bi_0),
              (W_gc_1, b_gc_1, W_bi_1, b_bi_1),
              (W_gc_2, b_gc_2, W_bi_2, b_bi_2)]
    ego = ego0
    norms = []
    for (wgc, bgc, wbi, bbi) in params:
        side = _spmm(ego, col, adj_val, row, bounds)
        ego, nrm = _dense(side, ego, wgc, bgc, wbi, bbi)
        norms.append(nrm)

    idx = jnp.concatenate([users.astype(jnp.int32),
                           pos_items.astype(jnp.int32) + NUSR,
                           neg_items.astype(jnp.int32) + NUSR])
    o0, o1, o2, o3 = _gather4(ego0, norms[0], norms[1], norms[2], idx)
    allg = jnp.concatenate([o0, o1, o2, o3], axis=1)
    return (allg[:1024], allg[1024:2048], allg[2048:])


# speedup vs baseline: 10.3324x; 2.9186x over previous
import functools

import jax
import jax.numpy as jnp
from jax import lax
from jax.experimental import pallas as pl
from jax.experimental.pallas import tpu as pltpu
from jax.experimental.pallas import tpu_sc as plsc

NUSR = 10000
NTOT = 50000
NEDGE = 800000
D = 64
NPAD = 51200
R_SC = NPAD // 2
NPASS = 2
R_ACC = R_SC // NPASS
NS = 16
NC = 2
R_TILE = R_ACC // NS
CH = 128
BE = 512
SBE = 2 * BE
EPAD = 800768
NBLK = EPAD // BE
ZROWS = 200
BGATH = 3072
GPW = BGATH // (NC * NS)

_mesh = plsc.VectorSubcoreMesh(
    core_axis_name="c", subcore_axis_name="s", num_cores=NC, num_subcores=NS)


@functools.partial(
    pl.kernel,
    out_type=jax.ShapeDtypeStruct((NPAD, D), jnp.float32),
    mesh=_mesh,
    scratch_types=[
        pltpu.VMEM_SHARED((R_ACC, D), jnp.float32),
        pltpu.VMEM((16,), jnp.int32),
        pltpu.VMEM((2, 4, CH), jnp.int32),
        pltpu.VMEM((2, 4, CH), jnp.int32),
        pltpu.VMEM((4, CH), jnp.float32),
        pltpu.VMEM((4, CH), jnp.float32),
        pltpu.VMEM((4, CH), jnp.int32),
        pltpu.VMEM((4, CH), jnp.int32),
        pltpu.VMEM((4, CH, D), jnp.float32),
        pltpu.VMEM((4, CH, D), jnp.float32),
        pltpu.SemaphoreType.DMA,
        pltpu.SemaphoreType.DMA,
        pltpu.SemaphoreType.DMA,
        pltpu.SemaphoreType.DMA,
    ],
    compiler_params=pltpu.CompilerParams(use_tc_tiling_on_sc=False),
)
def _spmm(ego_hbm, pk_hbm, val_hbm, bounds_hbm, out_hbm,
          acc, bbuf, idxA, idxB, valA, valB, rlocA, rlocB, gbA, gbB,
          semGA, semGB, semSA, semSB):
    c = lax.axis_index("c")
    s = lax.axis_index("s")
    pltpu.sync_copy(bounds_hbm, bbuf)
    bv = bbuf[pl.ds(0, 16)]
    iota16 = lax.iota(jnp.int32, 16)
    zero16 = jnp.zeros((16,), jnp.float32)

    def prep(idxr, valr, rlocr, blk, e_lo, e_hi, base_row):
        pltpu.sync_copy((pk_hbm.at[blk], val_hbm.at[blk]), (idxr, valr))
        ebase = blk * BE
        for g in range(4):
            for j in range(CH // 16):
                ids = ebase + (g * CH + j * 16) + iota16
                keep = (ids >= e_lo) & (ids < e_hi)
                vf = jnp.where(keep, valr[g, pl.ds(j * 16, 16)], 0.0)
                valr[g, pl.ds(j * 16, 16)] = vf
                rl = idxr[1, g, pl.ds(j * 16, 16)] - base_row
                rl = jnp.minimum(jnp.maximum(rl, 0), R_ACC - 1)
                rlocr[g, pl.ds(j * 16, 16)] = rl

    def g_descs(idxr, gbr, sem):
        return [pltpu.make_async_copy(ego_hbm.at[idxr.at[0, g]],
                                      gbr.at[g], sem) for g in range(4)]

    def s_descs(gbr, rlocr, sem):
        return [pltpu.make_async_copy(gbr.at[g], acc.at[rlocr.at[g]], sem)
                for g in range(4)]

    def scale(valr, gbr):
        def body(g, carry):
            for j in range(CH // 16):
                vv = valr[g, pl.ds(j * 16, 16)]
                for l in range(16):
                    v = vv[l]
                    e = j * 16 + l
                    for k in range(D // 16):
                        gbr[g, e, pl.ds(k * 16, 16)] = (
                            gbr[g, e, pl.ds(k * 16, 16)] * v)
            return carry

        lax.fori_loop(0, 4, body, 0)

    def pass_body(p, carry):
        i2 = c * NPASS + p
        e_lo = jnp.where(i2 == 0, bv[0],
                         jnp.where(i2 == 1, bv[1],
                                   jnp.where(i2 == 2, bv[2], bv[3])))
        e_hi = jnp.where(i2 == 0, bv[1],
                         jnp.where(i2 == 1, bv[2],
                                   jnp.where(i2 == 2, bv[3], bv[4])))
        base_row = i2 * R_ACC

        def zrow(r, carry2):
            for k in range(D // 16):
                gbB[0, r, pl.ds(k * 16, 16)] = zero16
            return carry2

        lax.fori_loop(0, CH, zrow, 0)
        for t in range(R_TILE // CH):
            pltpu.sync_copy(gbB.at[0],
                            acc.at[pl.ds(s * R_TILE + t * CH, CH)])
        rem_rows = R_TILE % CH
        if rem_rows:
            pltpu.sync_copy(
                gbB.at[0, pl.ds(0, rem_rows)],
                acc.at[pl.ds(s * R_TILE + (R_TILE // CH) * CH, rem_rows)])
        plsc.subcore_barrier()

        b0 = e_lo // SBE
        b1 = (e_hi + SBE - 1) // SBE
        nb = b1 - b0
        q = nb // NS
        rem = nb % NS
        start = b0 + s * q + jnp.minimum(s, rem)
        nsb = q + jnp.where(s < rem, 1, 0)

        @pl.when(nsb > 0)
        def _():
            prep(idxA, valA, rlocA, start * 2, e_lo, e_hi, base_row)
            for d_ in g_descs(idxA, gbA, semGA):
                d_.start()

        def sbody(i, carry2):
            bA = (start + i) * 2

            @pl.when(i > 0)
            def _():
                for d_ in s_descs(gbB, rlocB, semSB):
                    d_.wait()

            prep(idxB, valB, rlocB, bA + 1, e_lo, e_hi, base_row)
            for d_ in g_descs(idxB, gbB, semGB):
                d_.start()
            for d_ in g_descs(idxA, gbA, semGA):
                d_.wait()
            scale(valA, gbA)
            for d_ in s_descs(gbA, rlocA, semSA):
                d_.start(add=True)

            @pl.when(i + 1 < nsb)
            def _():
                for d_ in s_descs(gbA, rlocA, semSA):
                    d_.wait()
                prep(idxA, valA, rlocA, bA + 2, e_lo, e_hi, base_row)
                for d_ in g_descs(idxA, gbA, semGA):
                    d_.start()

            for d_ in g_descs(idxB, gbB, semGB):
                d_.wait()
            scale(valB, gbB)
            for d_ in s_descs(gbB, rlocB, semSB):
                d_.start(add=True)
            return carry2

        lax.fori_loop(0, nsb, sbody, 0)

        @pl.when(nsb > 0)
        def _():
            for d_ in s_descs(gbA, rlocA, semSA):
                d_.wait()
            for d_ in s_descs(gbB, rlocB, semSB):
                d_.wait()

        plsc.subcore_barrier()
        pltpu.sync_copy(acc.at[pl.ds(s * R_TILE, R_TILE)],
                        out_hbm.at[pl.ds(base_row + s * R_TILE, R_TILE)])
        return carry

    lax.fori_loop(0, NPASS, pass_body, 0)


def _dense_body(side_ref, ego_ref, wgc_ref, bgc_ref, wbi_ref, bbi_ref,
                next_ref, norm_ref):
    sd = side_ref[...]
    eg = ego_ref[...]
    t = jnp.dot(sd, wgc_ref[...], preferred_element_type=jnp.float32)
    t = t + jnp.dot(eg * sd, wbi_ref[...], preferred_element_type=jnp.float32)
    t = t + bgc_ref[...] + bbi_ref[...]
    a = jnp.where(t >= 0, t, 0.2 * t)
    next_ref[...] = a
    nrm = jnp.sqrt(jnp.sum(a * a, axis=1, keepdims=True))
    norm_ref[...] = a / jnp.maximum(nrm, 1e-12)


_DBLK = 1024


def _dense(side, ego, wgc, bgc, wbi, bbi):
    grid = (NPAD // _DBLK,)
    row_spec = pl.BlockSpec((_DBLK, D), lambda i: (i, 0))
    full = pl.BlockSpec((D, D), lambda i: (0, 0))
    bias = pl.BlockSpec((1, D), lambda i: (0, 0))
    return pl.pallas_call(
        _dense_body,
        grid=grid,
        in_specs=[row_spec, row_spec, full, bias, full, bias],
        out_specs=[row_spec, row_spec],
        out_shape=[jax.ShapeDtypeStruct((NPAD, D), jnp.float32),
                   jax.ShapeDtypeStruct((NPAD, D), jnp.float32)],
    )(side, ego, wgc, bgc, wbi, bbi)


@functools.partial(
    pl.kernel,
    out_type=[jax.ShapeDtypeStruct((BGATH, D), jnp.float32)] * 4,
    mesh=_mesh,
    scratch_types=[
        pltpu.VMEM((GPW,), jnp.int32),
        pltpu.VMEM((GPW, D), jnp.float32),
        pltpu.SemaphoreType.DMA,
    ],
    compiler_params=pltpu.CompilerParams(use_tc_tiling_on_sc=False),
)
def _gather4(t0, t1, t2, t3, idx_hbm, o0, o1, o2, o3, idxb, rows, sem):
    c = lax.axis_index("c")
    s = lax.axis_index("s")
    base = (s * NC + c) * GPW
    pltpu.sync_copy(idx_hbm.at[pl.ds(base, GPW)], idxb)
    for tbl, out in ((t0, o0), (t1, o1), (t2, o2), (t3, o3)):
        pltpu.async_copy(tbl.at[idxb], rows, sem).wait()
        pltpu.sync_copy(rows, out.at[pl.ds(base, GPW)])


def kernel(user_emb, item_emb,
           W_gc_0, b_gc_0, W_bi_0, b_bi_0,
           W_gc_1, b_gc_1, W_bi_1, b_bi_1,
           W_gc_2, b_gc_2, W_bi_2, b_bi_2,
           adj_row, adj_col, adj_val,
           users, pos_items, neg_items):
    ego0 = jnp.concatenate(
        [user_emb, item_emb,
         jnp.zeros((NPAD - NTOT, D), jnp.float32)], axis=0)
    row = adj_row.astype(jnp.int32)
    col = adj_col.astype(jnp.int32)
    splits = jnp.searchsorted(
        row, jnp.array([R_ACC, 2 * R_ACC, 3 * R_ACC], jnp.int32),
        side="left").astype(jnp.int32)
    bounds = jnp.zeros((16,), jnp.int32)
    bounds = bounds.at[1].set(splits[0]).at[2].set(splits[1])
    bounds = bounds.at[3].set(splits[2]).at[4].set(NEDGE)
    zpad = jnp.zeros((EPAD - NEDGE,), jnp.int32)
    pk = jnp.stack(
        [jnp.concatenate([col, zpad]).reshape(NBLK, 4, CH),
         jnp.concatenate([row, zpad]).reshape(NBLK, 4, CH)],
        axis=1)
    valp = jnp.concatenate(
        [adj_val, jnp.zeros((EPAD - NEDGE,), jnp.float32)]).reshape(
            NBLK, 4, CH)

    params = [(W_gc_0, b_gc_0, W_bi_0, b_bi_0),
              (W_gc_1, b_gc_1, W_bi_1, b_bi_1),
              (W_gc_2, b_gc_2, W_bi_2, b_bi_2)]
    ego = ego0
    norms = []
    for (wgc, bgc, wbi, bbi) in params:
        side = _spmm(ego, pk, valp, bounds)
        ego, nrm = _dense(side, ego, wgc, bgc, wbi, bbi)
        norms.append(nrm)

    idx = jnp.concatenate([users.astype(jnp.int32),
                           pos_items.astype(jnp.int32) + NUSR,
                           neg_items.astype(jnp.int32) + NUSR])
    o0, o1, o2, o3 = _gather4(ego0, norms[0], norms[1], norms[2], idx)
    allg = jnp.concatenate([o0, o1, o2, o3], axis=1)
    return (allg[:1024], allg[1024:2048], allg[2048:])


# speedup vs baseline: 11.0274x; 1.0673x over previous
import functools

import jax
import jax.numpy as jnp
from jax import lax
from jax.experimental import pallas as pl
from jax.experimental.pallas import tpu as pltpu
from jax.experimental.pallas import tpu_sc as plsc

NUSR = 10000
NTOT = 50000
NEDGE = 800000
D = 64
NPAD = 51200
R_SC = NPAD // 2
NPASS = 2
R_ACC = R_SC // NPASS
NS = 16
NC = 2
R_TILE = R_ACC // NS
CH = 128
BE = 512
SBE = 2 * BE
EPAD = 800768
NBLK = EPAD // BE
ZROWS = 200
BGATH = 3072
GPW = BGATH // (NC * NS)

_mesh = plsc.VectorSubcoreMesh(
    core_axis_name="c", subcore_axis_name="s", num_cores=NC, num_subcores=NS)


@functools.partial(
    pl.kernel,
    out_type=jax.ShapeDtypeStruct((NPAD, D), jnp.float32),
    mesh=_mesh,
    scratch_types=[
        pltpu.VMEM_SHARED((R_ACC, D), jnp.float32),
        pltpu.VMEM((16,), jnp.int32),
        pltpu.VMEM((2, 4, CH), jnp.int32),
        pltpu.VMEM((2, 4, CH), jnp.int32),
        pltpu.VMEM((4, CH), jnp.float32),
        pltpu.VMEM((4, CH), jnp.float32),
        pltpu.VMEM((2, 4, CH), jnp.int32),
        pltpu.VMEM((4, CH), jnp.int32),
        pltpu.VMEM((4, CH, D), jnp.float32),
        pltpu.VMEM((4, CH, D), jnp.float32),
        pltpu.SemaphoreType.DMA,
        pltpu.SemaphoreType.DMA,
        pltpu.SemaphoreType.DMA,
        pltpu.SemaphoreType.DMA,
    ],
    compiler_params=pltpu.CompilerParams(use_tc_tiling_on_sc=False),
)
def _spmm(ego_hbm, pk_hbm, val_hbm, bounds_hbm, out_hbm,
          acc, bbuf, idxA, idxB, valA, valB, rlocA, rlocB, gbA, gbB,
          semGA, semGB, semSA, semSB):
    c = lax.axis_index("c")
    s = lax.axis_index("s")
    pltpu.sync_copy(bounds_hbm, bbuf)
    bv = bbuf[pl.ds(0, 16)]
    iota16 = lax.iota(jnp.int32, 16)
    zero16 = jnp.zeros((16,), jnp.float32)

    def prep(idxr, valr, rlocr, blk, e_lo, e_hi, base_row):
        pltpu.sync_copy((pk_hbm.at[blk], val_hbm.at[blk]), (idxr, valr))
        ebase = blk * BE
        for g in range(4):
            for j in range(CH // 16):
                ids = ebase + (g * CH + j * 16) + iota16
                keep = (ids >= e_lo) & (ids < e_hi)
                vf = jnp.where(keep, valr[g, pl.ds(j * 16, 16)], 0.0)
                valr[g, pl.ds(j * 16, 16)] = vf
                rl = idxr[1, g, pl.ds(j * 16, 16)] - base_row
                rl = jnp.minimum(jnp.maximum(rl, 0), R_ACC - 1)
                rlocr[g, pl.ds(j * 16, 16)] = rl

    def g_descs(idxr, gbr, sem):
        return [pltpu.make_async_copy(ego_hbm.at[idxr.at[0, g]],
                                      gbr.at[g], sem) for g in range(4)]

    def s_descs(gbr, rlocr, sem):
        return [pltpu.make_async_copy(gbr.at[g], acc.at[rlocr.at[g]], sem)
                for g in range(4)]

    def scale(valr, gbr):
        def body(g, carry):
            for j in range(CH // 16):
                vv = valr[g, pl.ds(j * 16, 16)]
                for l in range(16):
                    v = vv[l]
                    e = j * 16 + l
                    for k in range(D // 16):
                        gbr[g, e, pl.ds(k * 16, 16)] = (
                            gbr[g, e, pl.ds(k * 16, 16)] * v)
            return carry

        lax.fori_loop(0, 4, body, 0)

    def pass_body(p, carry):
        i2 = c * NPASS + p
        e_lo = jnp.where(i2 == 0, bv[0],
                         jnp.where(i2 == 1, bv[1],
                                   jnp.where(i2 == 2, bv[2], bv[3])))
        e_hi = jnp.where(i2 == 0, bv[1],
                         jnp.where(i2 == 1, bv[2],
                                   jnp.where(i2 == 2, bv[3], bv[4])))
        base_row = i2 * R_ACC

        def zrow(r, carry2):
            for k in range(D // 16):
                gbB[0, r, pl.ds(k * 16, 16)] = zero16
            return carry2

        lax.fori_loop(0, CH, zrow, 0)
        for t in range(R_TILE // CH):
            pltpu.sync_copy(gbB.at[0],
                            acc.at[pl.ds(s * R_TILE + t * CH, CH)])
        rem_rows = R_TILE % CH
        if rem_rows:
            pltpu.sync_copy(
                gbB.at[0, pl.ds(0, rem_rows)],
                acc.at[pl.ds(s * R_TILE + (R_TILE // CH) * CH, rem_rows)])
        plsc.subcore_barrier()

        b0 = e_lo // SBE
        b1 = (e_hi + SBE - 1) // SBE
        nb = b1 - b0
        q = nb // NS
        rem = nb % NS
        start = b0 + s * q + jnp.minimum(s, rem)
        nsb = q + jnp.where(s < rem, 1, 0)

        @pl.when(nsb > 0)
        def _():
            prep(idxA, valA, rlocA.at[0], start * 2, e_lo, e_hi, base_row)
            for d_ in g_descs(idxA, gbA, semGA):
                d_.start()

        def sbody(i, carry2):
            bA = (start + i) * 2
            qA = lax.rem(i, 2)

            @pl.when(i > 0)
            def _():
                for d_ in s_descs(gbB, rlocB, semSB):
                    d_.wait()

            prep(idxB, valB, rlocB, bA + 1, e_lo, e_hi, base_row)
            for d_ in g_descs(idxB, gbB, semGB):
                d_.start()
            for d_ in g_descs(idxA, gbA, semGA):
                d_.wait()
            scale(valA, gbA)
            for d_ in s_descs(gbA, rlocA.at[qA], semSA):
                d_.start(add=True)

            @pl.when(i + 1 < nsb)
            def _():
                prep(idxA, valA, rlocA.at[1 - qA], bA + 2,
                     e_lo, e_hi, base_row)
                for d_ in s_descs(gbA, rlocA.at[qA], semSA):
                    d_.wait()
                for d_ in g_descs(idxA, gbA, semGA):
                    d_.start()

            for d_ in g_descs(idxB, gbB, semGB):
                d_.wait()
            scale(valB, gbB)
            for d_ in s_descs(gbB, rlocB, semSB):
                d_.start(add=True)
            return carry2

        lax.fori_loop(0, nsb, sbody, 0)

        @pl.when(nsb > 0)
        def _():
            for d_ in s_descs(gbA, rlocA.at[0], semSA):
                d_.wait()
            for d_ in s_descs(gbB, rlocB, semSB):
                d_.wait()

        plsc.subcore_barrier()
        pltpu.sync_copy(acc.at[pl.ds(s * R_TILE, R_TILE)],
                        out_hbm.at[pl.ds(base_row + s * R_TILE, R_TILE)])
        return carry

    lax.fori_loop(0, NPASS, pass_body, 0)


def _dense_body(side_ref, ego_ref, wgc_ref, bgc_ref, wbi_ref, bbi_ref,
                next_ref):
    sd = side_ref[...]
    eg = ego_ref[...]
    t = jnp.dot(sd, wgc_ref[...], preferred_element_type=jnp.float32)
    t = t + jnp.dot(eg * sd, wbi_ref[...], preferred_element_type=jnp.float32)
    t = t + bgc_ref[...] + bbi_ref[...]
    next_ref[...] = jnp.where(t >= 0, t, 0.2 * t)


_DBLK = 1024


def _dense(side, ego, wgc, bgc, wbi, bbi):
    grid = (NPAD // _DBLK,)
    row_spec = pl.BlockSpec((_DBLK, D), lambda i: (i, 0))
    full = pl.BlockSpec((D, D), lambda i: (0, 0))
    bias = pl.BlockSpec((1, D), lambda i: (0, 0))
    return pl.pallas_call(
        _dense_body,
        grid=grid,
        in_specs=[row_spec, row_spec, full, bias, full, bias],
        out_specs=row_spec,
        out_shape=jax.ShapeDtypeStruct((NPAD, D), jnp.float32),
    )(side, ego, wgc, bgc, wbi, bbi)


@functools.partial(
    pl.kernel,
    out_type=[jax.ShapeDtypeStruct((BGATH, D), jnp.float32)] * 4,
    mesh=_mesh,
    scratch_types=[
        pltpu.VMEM((GPW,), jnp.int32),
        pltpu.VMEM((GPW, D), jnp.float32),
        pltpu.SemaphoreType.DMA,
    ],
    compiler_params=pltpu.CompilerParams(use_tc_tiling_on_sc=False),
)
def _gather4(t0, t1, t2, t3, idx_hbm, o0, o1, o2, o3, idxb, rows, sem):
    c = lax.axis_index("c")
    s = lax.axis_index("s")
    base = (s * NC + c) * GPW
    pltpu.sync_copy(idx_hbm.at[pl.ds(base, GPW)], idxb)
    for tbl, out in ((t0, o0), (t1, o1), (t2, o2), (t3, o3)):
        pltpu.async_copy(tbl.at[idxb], rows, sem).wait()
        pltpu.sync_copy(rows, out.at[pl.ds(base, GPW)])


def _gnorm_body(x_ref, o_ref):
    x = x_ref[...]
    pieces = [x[:, 0:D]]
    for t in range(1, 4):
        a = x[:, t * D:(t + 1) * D]
        nrm = jnp.sqrt(jnp.sum(a * a, axis=1, keepdims=True))
        pieces.append(a / jnp.maximum(nrm, 1e-12))
    o_ref[...] = jnp.concatenate(pieces, axis=1)


def _gnorm(x):
    blk = pl.BlockSpec((BGATH, 4 * D), lambda: (0, 0))
    return pl.pallas_call(
        _gnorm_body,
        in_specs=[blk],
        out_specs=blk,
        out_shape=jax.ShapeDtypeStruct((BGATH, 4 * D), jnp.float32),
    )(x)


def _searchsorted3(row):
    tgt = jnp.array([R_ACC, 2 * R_ACC, 3 * R_ACC], jnp.int32)
    lo = jnp.zeros((3,), jnp.int32)
    hi = jnp.full((3,), NEDGE, jnp.int32)

    def step(_, lh):
        lo, hi = lh
        mid = (lo + hi) // 2
        less = jnp.take(row, mid) < tgt
        return (jnp.where(less, mid + 1, lo), jnp.where(less, hi, mid))

    lo, hi = lax.fori_loop(0, 20, step, (lo, hi))
    return lo


def kernel(user_emb, item_emb,
           W_gc_0, b_gc_0, W_bi_0, b_bi_0,
           W_gc_1, b_gc_1, W_bi_1, b_bi_1,
           W_gc_2, b_gc_2, W_bi_2, b_bi_2,
           adj_row, adj_col, adj_val,
           users, pos_items, neg_items):
    ego0 = jnp.concatenate(
        [user_emb, item_emb,
         jnp.zeros((NPAD - NTOT, D), jnp.float32)], axis=0)
    row = adj_row.astype(jnp.int32)
    col = adj_col.astype(jnp.int32)
    splits = _searchsorted3(row)
    bounds = jnp.zeros((16,), jnp.int32)
    bounds = bounds.at[1].set(splits[0]).at[2].set(splits[1])
    bounds = bounds.at[3].set(splits[2]).at[4].set(NEDGE)
    zpad = jnp.zeros((EPAD - NEDGE,), jnp.int32)
    pk = jnp.stack(
        [jnp.concatenate([col, zpad]).reshape(NBLK, 4, CH),
         jnp.concatenate([row, zpad]).reshape(NBLK, 4, CH)],
        axis=1)
    valp = jnp.concatenate(
        [adj_val, jnp.zeros((EPAD - NEDGE,), jnp.float32)]).reshape(
            NBLK, 4, CH)

    params = [(W_gc_0, b_gc_0, W_bi_0, b_bi_0),
              (W_gc_1, b_gc_1, W_bi_1, b_bi_1),
              (W_gc_2, b_gc_2, W_bi_2, b_bi_2)]
    ego = ego0
    egos = []
    for (wgc, bgc, wbi, bbi) in params:
        side = _spmm(ego, pk, valp, bounds)
        ego = _dense(side, ego, wgc, bgc, wbi, bbi)
        egos.append(ego)

    idx = jnp.concatenate([users.astype(jnp.int32),
                           pos_items.astype(jnp.int32) + NUSR,
                           neg_items.astype(jnp.int32) + NUSR])
    o0, o1, o2, o3 = _gather4(ego0, egos[0], egos[1], egos[2], idx)
    allg = _gnorm(jnp.concatenate([o0, o1, o2, o3], axis=1))
    return (allg[:1024], allg[1024:2048], allg[2048:])
